# Initial kernel scaffold; baseline (speedup 1.0000x reference)
import functools

import jax
import jax.numpy as jnp
from jax import lax
from jax.experimental import pallas as pl
from jax.experimental.pallas import tpu as pltpu
from jax.experimental.pallas import tpu_sc as plsc

_NC = 2
_NS = 16
_NW = _NC * _NS
_L = 16

_ROWS = 8
_TILE = _ROWS * 128


def _render_call(faces, attributes, p2f_r, bary_r):
    nrows, _ = p2f_r.shape
    D = attributes.shape[1]
    rows_per_w = nrows // _NW
    ntiles = rows_per_w // _ROWS

    mesh = plsc.VectorSubcoreMesh(core_axis_name="c", subcore_axis_name="s")

    @functools.partial(
        pl.kernel,
        out_type=(
            jax.ShapeDtypeStruct((nrows, 128, D), jnp.float32),
            jax.ShapeDtypeStruct((nrows, 128), jnp.int32),
        ),
        mesh=mesh,
        scratch_types=[
            pltpu.VMEM((_ROWS, 128), jnp.int32),
            pltpu.VMEM((_ROWS, 128, 3), jnp.float32),
            pltpu.VMEM((_ROWS, 128, 3), jnp.int32),
            pltpu.VMEM((3 * _ROWS, 128), jnp.int32),
            pltpu.VMEM((3 * _ROWS, 128, D), jnp.float32),
            pltpu.VMEM((_ROWS, 128, D), jnp.float32),
            pltpu.VMEM((_ROWS, 128), jnp.int32),
            pltpu.SemaphoreType.DMA,
        ],
    )
    def render(faces_h, attr_h, p2f_h, bary_h, out_h, mask_h,
               p2f_v, bary_v, vidx_v, vcol_v, attr_v, out_v, mask_v, sem):
        wid = lax.axis_index("s") * _NC + lax.axis_index("c")
        base_row = wid * rows_per_w
        lanes = lax.iota(jnp.int32, _L)

        def tile_body(t, carry):
            r0 = base_row + t * _ROWS
            pltpu.sync_copy(p2f_h.at[pl.ds(r0, _ROWS)], p2f_v)
            pltpu.sync_copy(bary_h.at[pl.ds(r0, _ROWS)], bary_v)

            cps = [pltpu.async_copy(faces_h.at[p2f_v.at[j]], vidx_v.at[j], sem)
                   for j in range(_ROWS)]
            for cp in cps:
                cp.wait()

            for k in range(3):
                ksplat = jnp.full((_L,), k, jnp.int32)
                for j in range(_ROWS):
                    for g in range(128 // _L):
                        vec = plsc.load_gather(
                            vidx_v.at[j], [lanes + (g * _L), ksplat])
                        vcol_v[k * _ROWS + j, pl.ds(g * _L, _L)] = vec

            cps = [pltpu.async_copy(attr_h.at[vcol_v.at[i]], attr_v.at[i], sem)
                   for i in range(3 * _ROWS)]
            for cp in cps:
                cp.wait()

            for j in range(_ROWS):
                def px_body(p, c, j=j):
                    b0 = bary_v[j, p, 0]
                    b1 = bary_v[j, p, 1]
                    b2 = bary_v[j, p, 2]
                    a0 = attr_v[0 * _ROWS + j, p, :]
                    a1 = attr_v[1 * _ROWS + j, p, :]
                    a2 = attr_v[2 * _ROWS + j, p, :]
                    out_v[j, p, :] = a0 * b0 + a1 * b1 + a2 * b2
                    return c
                lax.fori_loop(0, 128, px_body, 0, unroll=4)

            one = jnp.full((_L,), 1, jnp.int32)
            zero = jnp.full((_L,), 0, jnp.int32)
            for j in range(_ROWS):
                for g in range(128 // _L):
                    v = p2f_v[j, pl.ds(g * _L, _L)]
                    mask_v[j, pl.ds(g * _L, _L)] = jnp.where(
                        v != -1, one, zero)

            pltpu.sync_copy(out_v, out_h.at[pl.ds(r0, _ROWS)])
            pltpu.sync_copy(mask_v, mask_h.at[pl.ds(r0, _ROWS)])
            return carry

        lax.fori_loop(0, ntiles, tile_body, 0)

    return render(faces, attributes, p2f_r, bary_r)


def kernel(vertices, faces, attributes, pix_to_face, bary_coords):
    H, W = pix_to_face.shape
    N = H * W
    D = attributes.shape[1]
    p2f_r = pix_to_face.reshape(N // 128, 128)
    bary_r = bary_coords.reshape(N // 128, 128, 3)
    out, mask_i = _render_call(faces, attributes, p2f_r, bary_r)
    attribute_map = out.reshape(H, W, D)
    mask = mask_i.reshape(H, W).astype(bool)
    return (attribute_map, mask)


# speedup vs baseline: 15.3457x; 15.3457x over previous
import functools

import jax
import jax.numpy as jnp
from jax import lax
from jax.experimental import pallas as pl
from jax.experimental.pallas import tpu as pltpu
from jax.experimental.pallas import tpu_sc as plsc

_NC = 2
_NS = 16
_NW = _NC * _NS
_L = 16

_ROWS = 8
_TILE = _ROWS * 128


def _render_call(faces, attributes, p2f_r, bary_r):
    nrows, _ = p2f_r.shape
    D = attributes.shape[1]
    rows_per_w = nrows // _NW
    ntiles = rows_per_w // _ROWS

    mesh = plsc.VectorSubcoreMesh(core_axis_name="c", subcore_axis_name="s")

    @functools.partial(
        pl.kernel,
        out_type=(
            jax.ShapeDtypeStruct((nrows, 128, D), jnp.float32),
            jax.ShapeDtypeStruct((nrows, 128), jnp.int32),
        ),
        mesh=mesh,
        compiler_params=pltpu.CompilerParams(use_tc_tiling_on_sc=False),
        scratch_types=[
            pltpu.VMEM((_ROWS, 128), jnp.int32),
            pltpu.VMEM((_ROWS, 3, 128), jnp.float32),
            pltpu.VMEM((3 * _ROWS, 128), jnp.int32),
            pltpu.VMEM((3 * _ROWS, 128), jnp.int32),
            pltpu.VMEM((3 * _ROWS, 128, D), jnp.float32),
            pltpu.VMEM((_ROWS, 128, D), jnp.float32),
            pltpu.VMEM((_ROWS, 128), jnp.int32),
            pltpu.SemaphoreType.DMA,
        ],
    )
    def render(faces_h, attr_h, p2f_h, bary_h, out_h, mask_h,
               p2f_v, bary_v, fidx_v, vert_v, attr_v, out_v, mask_v, sem):
        wid = lax.axis_index("s") * _NC + lax.axis_index("c")
        base_row = wid * rows_per_w
        lanes = lax.iota(jnp.int32, _L)

        def tile_body(t, carry):
            r0 = base_row + t * _ROWS
            pltpu.sync_copy(p2f_h.at[pl.ds(r0, _ROWS)], p2f_v)
            pltpu.sync_copy(bary_h.at[pl.ds(r0, _ROWS)], bary_v)

            for j in range(_ROWS):
                for g in range(128 // _L):
                    sl = pl.ds(g * _L, _L)
                    t = p2f_v[j, sl] * 3
                    fidx_v[0 * _ROWS + j, sl] = t
                    fidx_v[1 * _ROWS + j, sl] = t + 1
                    fidx_v[2 * _ROWS + j, sl] = t + 2

            cps = [pltpu.async_copy(faces_h.at[fidx_v.at[i]], vert_v.at[i],
                                    sem)
                   for i in range(3 * _ROWS)]
            for cp in cps:
                cp.wait()

            cps = [pltpu.async_copy(attr_h.at[vert_v.at[i]], attr_v.at[i],
                                    sem)
                   for i in range(3 * _ROWS)]
            for cp in cps:
                cp.wait()

            for j in range(_ROWS):
                def g_body(g, c, j=j):
                    base = g * _L
                    b0v = bary_v[j, 0, pl.ds(base, _L)]
                    b1v = bary_v[j, 1, pl.ds(base, _L)]
                    b2v = bary_v[j, 2, pl.ds(base, _L)]
                    for l in range(_L):
                        p = base + l
                        acc = (attr_v[0 * _ROWS + j, p, :] * b0v[l]
                               + attr_v[1 * _ROWS + j, p, :] * b1v[l]
                               + attr_v[2 * _ROWS + j, p, :] * b2v[l])
                        out_v[j, p, :] = acc
                    return c
                lax.fori_loop(0, 128 // _L, g_body, 0)

            one = jnp.full((_L,), 1, jnp.int32)
            zero = jnp.full((_L,), 0, jnp.int32)
            for j in range(_ROWS):
                for g in range(128 // _L):
                    v = p2f_v[j, pl.ds(g * _L, _L)]
                    mask_v[j, pl.ds(g * _L, _L)] = jnp.where(
                        v != -1, one, zero)

            pltpu.sync_copy(out_v, out_h.at[pl.ds(r0, _ROWS)])
            pltpu.sync_copy(mask_v, mask_h.at[pl.ds(r0, _ROWS)])
            return carry

        lax.fori_loop(0, ntiles, tile_body, 0)

    return render(faces, attributes, p2f_r, bary_r)


def kernel(vertices, faces, attributes, pix_to_face, bary_coords):
    H, W = pix_to_face.shape
    N = H * W
    D = attributes.shape[1]
    p2f_r = pix_to_face.reshape(N // 128, 128)
    bary_r = bary_coords.reshape(N // 128, 128, 3).transpose(0, 2, 1)
    faces_flat = faces.reshape(faces.shape[0] * 3)
    out, mask_i = _render_call(faces_flat, attributes, p2f_r, bary_r)
    attribute_map = out.reshape(H, W, D)
    mask = mask_i.reshape(H, W).astype(bool)
    return (attribute_map, mask)
